# baseline (speedup 1.0000x reference)
import functools

import jax
import jax.numpy as jnp
from jax import lax
from jax.experimental import pallas as pl
from jax.experimental.pallas import tpu as pltpu
from jax.experimental.pallas import tpu_sc as plsc

B = 1024
D = 64
T = 51200
V = 100000
RB = 4000
NC = 2
NS = 16
NW = NC * NS
TPW = T // NW
NPW = B // NW
GC = 80
NCH = TPW // GC
NSLOT = 4
CT = 64
KB = 2048
TPAD = T + KB
CUP = 1048

_mesh = plsc.VectorSubcoreMesh(core_axis_name="c", subcore_axis_name="s")
_sc_params = pltpu.CompilerParams(needs_layout_passes=False,
                                  use_tc_tiling_on_sc=False)
_sc_params_tiled = pltpu.CompilerParams(needs_layout_passes=False,
                                        use_tc_tiling_on_sc=True)


def _iota16():
    return lax.broadcasted_iota(jnp.int32, (16,), 0)


VCH = 160
NVCH = V // VCH
NVR = (NVCH + NW - 1) // NW


@functools.partial(
    pl.kernel,
    out_type=jax.ShapeDtypeStruct((V // 2, 2 * D), jnp.float32),
    mesh=_mesh,
    compiler_params=_sc_params_tiled,
    scratch_types=[
        [pltpu.VMEM((VCH, D), jnp.float32)] * 2,
        [pltpu.VMEM((VCH // 2, 2 * D), jnp.float32)] * 2,
        [pltpu.SemaphoreType.DMA] * 2,
        [pltpu.SemaphoreType.DMA] * 2,
    ],
)
def _tablin_stage(table_hbm, out_hbm, bin_, bout, sem_i, sem_o):
    wid = lax.axis_index("s") * NC + lax.axis_index("c")

    def start_in(i):
        c = wid + i * NW

        @pl.when(c < NVCH)
        def _():
            pltpu.async_copy(table_hbm.at[pl.ds(c * VCH, VCH)],
                             bin_[i % 2], sem_i[i % 2])

    start_in(0)
    for i in range(NVR):
        c = wid + i * NW
        s = i % 2

        @pl.when(c < NVCH)
        def _():
            pltpu.make_async_copy(table_hbm.at[pl.ds(c * VCH, VCH)],
                                  bin_[s], sem_i[s]).wait()

        if i + 1 < NVR:
            start_in(i + 1)

        @pl.when(c < NVCH)
        def _():
            if i >= 2:
                po = (wid + (i - 2) * NW) * (VCH // 2)
                pltpu.make_async_copy(
                    bout[s], out_hbm.at[pl.ds(po, VCH // 2)], sem_o[s]).wait()

            def rp(p, _):
                for h in range(2):
                    for q in range(D // 16):
                        bout[s][p, pl.ds(h * D + q * 16, 16)] = (
                            bin_[s][2 * p + h, pl.ds(q * 16, 16)])
                return 0

            lax.fori_loop(0, VCH // 2, rp, 0)
            pltpu.async_copy(bout[s],
                             out_hbm.at[pl.ds(c * (VCH // 2), VCH // 2)],
                             sem_o[s])

    for i in range(max(NVR - 2, 0), NVR):
        c = wid + i * NW
        s = i % 2

        @pl.when(c < NVCH)
        def _():
            pltpu.make_async_copy(
                bout[s], out_hbm.at[pl.ds(c * (VCH // 2), VCH // 2)],
                sem_o[s]).wait()


@functools.partial(
    pl.kernel,
    out_type=jax.ShapeDtypeStruct((TPAD, 2 * D), jnp.bfloat16),
    mesh=_mesh,
    compiler_params=_sc_params,
    scratch_types=[
        pltpu.VMEM((CUP,), jnp.int32),
        pltpu.VMEM((B,), jnp.int32),
        pltpu.VMEM((TPW,), jnp.int32),
        pltpu.VMEM((TPW,), jnp.int32),
        [pltpu.VMEM((GC, D), jnp.bfloat16)] * NSLOT,
        [pltpu.VMEM((GC, D), jnp.bfloat16)] * NSLOT,
        [pltpu.SemaphoreType.DMA] * NSLOT,
        [pltpu.SemaphoreType.DMA] * NSLOT,
        [pltpu.SemaphoreType.DMA] * NSLOT,
        [pltpu.SemaphoreType.DMA] * NSLOT,
    ],
)
def _gather_stage(cu_hbm, nodes_hbm, fn_hbm, table_hbm, out_hbm,
                  cu_v, nodes_v, fn_v, idx2_v, rows_n, rows_u,
                  gsem_n, gsem_u, wsem_n, wsem_u):
    wid = lax.axis_index("s") * NC + lax.axis_index("c")
    base = wid * TPW
    pltpu.sync_copy(cu_hbm, cu_v)
    pltpu.sync_copy(nodes_hbm, nodes_v)
    pltpu.sync_copy(fn_hbm.at[pl.ds(base, TPW)], fn_v)

    def bisect16(t, lo, hi):
        def cond(lh):
            return jnp.max(lh[1] - lh[0]) > 0

        def step(lh):
            lo_, hi_ = lh
            mid = lax.shift_right_logical(lo_ + hi_, 1)
            a = plsc.load_gather(cu_v, [mid + 1])
            p = a <= t
            return jnp.where(p, mid + 1, lo_), jnp.where(p, hi_, mid)

        lo, hi = lax.while_loop(cond, step, (lo, hi))
        return lo

    tlast = jnp.full((16,), base + TPW - 1, jnp.int32)
    hi0v = bisect16(tlast, jnp.zeros((16,), jnp.int32),
                    jnp.full((16,), B - 1, jnp.int32))
    hi0 = hi0v[0]

    def start_n(k):
        off = k * GC
        pltpu.async_copy(table_hbm.at[fn_v.at[pl.ds(off, GC)]],
                         rows_n[k % NSLOT], gsem_n[k % NSLOT])

    def start_u(k):
        off = k * GC
        pltpu.async_copy(table_hbm.at[idx2_v.at[pl.ds(off, GC)]],
                         rows_u[k % NSLOT], gsem_u[k % NSLOT])

    def wait_writes(k):
        s = k % NSLOT
        off = k * GC
        pltpu.make_async_copy(rows_n[s],
                              out_hbm.at[pl.ds(base + off, GC), pl.ds(0, D)],
                              wsem_n[s]).wait()
        pltpu.make_async_copy(rows_u[s],
                              out_hbm.at[pl.ds(base + off, GC), pl.ds(D, D)],
                              wsem_u[s]).wait()

    def finish(k):
        s = k % NSLOT
        off = k * GC
        pltpu.make_async_copy(table_hbm.at[fn_v.at[pl.ds(off, GC)]],
                              rows_n[s], gsem_n[s]).wait()
        pltpu.make_async_copy(table_hbm.at[idx2_v.at[pl.ds(off, GC)]],
                              rows_u[s], gsem_u[s]).wait()
        pltpu.async_copy(rows_n[s],
                         out_hbm.at[pl.ds(base + off, GC), pl.ds(0, D)],
                         wsem_n[s])
        pltpu.async_copy(rows_u[s],
                         out_hbm.at[pl.ds(base + off, GC), pl.ds(D, D)],
                         wsem_u[s])

    cur = jnp.int32(0)
    for k in range(NCH):
        if k >= NSLOT:
            wait_writes(k - NSLOT)
        start_n(k)
        for gg in range(GC // 16):
            t = base + (k * (GC // 16) + gg) * 16 + _iota16()
            lo = bisect16(t, jnp.full((16,), cur, jnp.int32),
                          jnp.full((16,), hi0, jnp.int32))
            idx2_v[pl.ds(k * GC + gg * 16, 16)] = (
                plsc.load_gather(nodes_v, [lo]))
            cur = lo[15]
        start_u(k)
        if k >= 1:
            finish(k - 1)
    finish(NCH - 1)
    for k in range(max(NCH - NSLOT, 0), NCH):
        wait_writes(k)


_DNT = (((0,), (1,)), ((), ()))
_DN0 = (((0,), (0,)), ((), ()))


def _mlp_body(x_ref, w1_ref, b1_ref, w2_ref, b2_ref, w3_ref, out_ref):
    bf = jnp.bfloat16
    h = lax.dot_general(w1_ref[...].astype(bf), x_ref[...],
                        _DNT, preferred_element_type=jnp.float32)
    h = jnp.maximum(h + b1_ref[...], 0.0)
    h = lax.dot_general(w2_ref[...].astype(bf), h.astype(bf),
                        _DN0, preferred_element_type=jnp.float32)
    h = jnp.maximum(h + b2_ref[...], 0.0)
    lg = lax.dot_general(w3_ref[...].astype(bf), h.astype(bf),
                         _DN0, preferred_element_type=jnp.float32)
    out_ref[...] = lg[0]


_mlp_call = pl.pallas_call(
    _mlp_body,
    grid=(TPAD // KB,),
    in_specs=[
        pl.BlockSpec((KB, 2 * D), lambda i: (i, 0)),
        pl.BlockSpec((2 * D, D), lambda i: (0, 0)),
        pl.BlockSpec((D, 1), lambda i: (0, 0)),
        pl.BlockSpec((D, D), lambda i: (0, 0)),
        pl.BlockSpec((D, 1), lambda i: (0, 0)),
        pl.BlockSpec((D, 1), lambda i: (0, 0)),
    ],
    out_specs=pl.BlockSpec((KB,), lambda i: (i,)),
    out_shape=jax.ShapeDtypeStruct((TPAD,), jnp.float32),
)


def _bf16_cols(v32):
    u = plsc.bitcast(v32, jnp.uint32)
    ev = plsc.bitcast(lax.shift_left(u, jnp.uint32(16)), jnp.float32)
    od = plsc.bitcast(u & jnp.uint32(0xFFFF0000), jnp.float32)
    return ev, od


@functools.partial(
    pl.kernel,
    out_type=jax.ShapeDtypeStruct((B, D), jnp.float32),
    mesh=_mesh,
    compiler_params=_sc_params,
    scratch_types=[
        pltpu.VMEM((48,), jnp.int32),
        [pltpu.VMEM((CT,), jnp.float32)] * 2,
        pltpu.VMEM((CT,), jnp.float32),
        [pltpu.VMEM((CT, D), jnp.bfloat16)] * 2,
        pltpu.VMEM((NPW, D), jnp.float32),
        pltpu.SMEM((4,), jnp.int32),
        pltpu.SMEM((4,), jnp.float32),
        pltpu.VMEM((5, 16), jnp.float32),
        [pltpu.SemaphoreType.DMA] * 2,
        [pltpu.SemaphoreType.DMA] * 2,
    ],
)
def _reduce_stage(cu_hbm, lg_hbm, en_hbm, out_hbm,
                  cu_v, lg_v, w_v, rows_v, out_v, si, sf, vs, sem_l, sem_r):
    wid = lax.axis_index("s") * NC + lax.axis_index("c")
    nbase = wid * NPW
    pltpu.sync_copy(cu_hbm.at[pl.ds(nbase, 48)], cu_v)
    neg = jnp.float32(-jnp.inf)
    zero = jnp.zeros((16,), jnp.float32)

    head = cu_v[pl.ds(0, 16)]
    tail = cu_v[pl.ds(NPW, 16)]
    s0 = head[0]
    big_e = tail[0]
    s8 = pl.multiple_of(lax.shift_left(lax.shift_right_logical(s0, 3), 3), 8)
    n_ch = lax.shift_right_logical(big_e - s8 + (CT - 1), 6)

    si[0] = 0
    si[1] = s0
    si[2] = head[1]
    sf[0] = neg
    for i in range(5):
        vs[i, pl.ds(0, 16)] = zero

    def start_dma(k, slot):
        g = pl.multiple_of(s8 + k * CT, 8)
        pltpu.async_copy(lg_hbm.at[pl.ds(g, CT)], lg_v[slot], sem_l[slot])
        pltpu.async_copy(en_hbm.at[pl.ds(g, CT), pl.ds(0, D)],
                         rows_v[slot], sem_r[slot])

    def wait_dma(k, slot):
        g = pl.multiple_of(s8 + k * CT, 8)
        pltpu.make_async_copy(lg_hbm.at[pl.ds(g, CT)], lg_v[slot],
                              sem_l[slot]).wait()
        pltpu.make_async_copy(en_hbm.at[pl.ds(g, CT), pl.ds(0, D)],
                              rows_v[slot], sem_r[slot]).wait()

    @pl.when(n_ch > 0)
    def _():
        start_dma(0, 0)

    def process(k, slot):
        g = pl.multiple_of(s8 + k * CT, 8)
        gend = g + CT

        def cond(c):
            return c[9] != 0

        def body(c):
            b, s, e, m, ssumv, a0, a1, a2, a3, _ = c
            msub = jnp.full((16,), neg, jnp.float32)
            lvs = []
            msks = []
            for q in range(CT // 16):
                gidx = g + q * 16 + _iota16()
                msk = (gidx >= s) & (gidx < e)
                lv = lg_v[slot][pl.ds(q * 16, 16)]
                lvs.append(lv)
                msks.append(msk)
                msub = jnp.maximum(msub, jnp.where(msk, lv, neg))
            mnew = jnp.maximum(m, jnp.max(msub))
            scale = jnp.exp(jnp.full((16,), m - mnew, jnp.float32))
            ssumv = ssumv * scale
            for q in range(CT // 16):
                wv = jnp.where(msks[q], jnp.exp(lvs[q] - mnew), 0.0)
                w_v[pl.ds(q * 16, 16)] = wv
                ssumv = ssumv + wv
            a0 = a0 * scale
            a1 = a1 * scale
            a2 = a2 * scale
            a3 = a3 * scale

            def tok(j, acc):
                t0, t1, t2, t3 = acc
                wj = plsc.load_gather(w_v, [jnp.full((16,), j, jnp.int32)])
                e0, o0 = _bf16_cols(rows_v[slot][j, pl.ds(0, 32)])
                e1, o1 = _bf16_cols(rows_v[slot][j, pl.ds(32, 32)])
                t0 = t0 + wj * e0
                t1 = t1 + wj * o0
                t2 = t2 + wj * e1
                t3 = t3 + wj * o1
                return t0, t1, t2, t3

            jlo = jnp.maximum(s - g, 0)
            jhi = jnp.minimum(e - g, CT)
            a0, a1, a2, a3 = lax.fori_loop(jlo, jhi, tok, (a0, a1, a2, a3))

            fin = e <= gend

            @pl.when(fin)
            def _():
                total = jnp.sum(ssumv)
                ok = total > 0.0
                bi = jnp.full((16,), b, jnp.int32)
                i2 = 2 * _iota16()
                plsc.store_scatter(out_v, [bi, i2],
                                   jnp.where(ok, a0 / total, 0.0))
                plsc.store_scatter(out_v, [bi, i2 + 1],
                                   jnp.where(ok, a1 / total, 0.0))
                plsc.store_scatter(out_v, [bi, i2 + 32],
                                   jnp.where(ok, a2 / total, 0.0))
                plsc.store_scatter(out_v, [bi, i2 + 33],
                                   jnp.where(ok, a3 / total, 0.0))

            b2 = jnp.where(fin, b + 1, b)
            pair = cu_v[pl.ds(b2, 16)]
            s2 = jnp.where(fin, pair[0], s)
            e2 = jnp.where(fin, pair[1], e)
            m2 = jnp.where(fin, neg, mnew)
            ssumv2 = jnp.where(fin, zero, ssumv)
            a02 = jnp.where(fin, zero, a0)
            a12 = jnp.where(fin, zero, a1)
            a22 = jnp.where(fin, zero, a2)
            a32 = jnp.where(fin, zero, a3)
            cont = jnp.where(fin & (b2 < NPW) & (s2 < gend),
                             jnp.int32(1), jnp.int32(0))
            return b2, s2, e2, m2, ssumv2, a02, a12, a22, a32, cont

        state = (si[0], si[1], si[2], sf[0],
                 vs[0, pl.ds(0, 16)], vs[1, pl.ds(0, 16)],
                 vs[2, pl.ds(0, 16)], vs[3, pl.ds(0, 16)],
                 vs[4, pl.ds(0, 16)], jnp.int32(1))
        b, s, e, m, ssumv, a0, a1, a2, a3, _ = lax.while_loop(
            cond, body, state)
        si[0] = b
        si[1] = s
        si[2] = e
        sf[0] = m
        vs[0, pl.ds(0, 16)] = ssumv
        vs[1, pl.ds(0, 16)] = a0
        vs[2, pl.ds(0, 16)] = a1
        vs[3, pl.ds(0, 16)] = a2
        vs[4, pl.ds(0, 16)] = a3

    def pair_body(i, _):
        k0 = i * 2
        for off in range(2):
            k = k0 + off
            slot = off

            @pl.when(k < n_ch)
            def _():
                wait_dma(k, slot)

                @pl.when(k + 1 < n_ch)
                def _():
                    start_dma(k + 1, 1 - slot)

                process(k, slot)
        return 0

    lax.fori_loop(0, lax.shift_right_logical(n_ch + 1, 1), pair_body, 0)

    def drain(b, _):
        out_v[b, pl.ds(0, 16)] = zero
        out_v[b, pl.ds(16, 16)] = zero
        out_v[b, pl.ds(32, 16)] = zero
        out_v[b, pl.ds(48, 16)] = zero
        return 0

    lax.fori_loop(si[0], NPW, drain, 0)
    pltpu.sync_copy(out_v, out_hbm.at[pl.ds(nbase, NPW)])


def kernel(nodes, flat_neighs, cu_seqlens, table, w1, b1, w2, b2, w3, b3):
    del b3
    cu_pad = jnp.full((CUP,), T, jnp.int32)
    cu_pad = lax.dynamic_update_slice(cu_pad, cu_seqlens.astype(jnp.int32),
                                      (0,))
    x = _gather_stage(cu_pad, nodes.astype(jnp.int32),
                      flat_neighs.astype(jnp.int32),
                      table.astype(jnp.bfloat16))
    logits = _mlp_call(x, w1, b1.reshape(D, 1), w2, b2.reshape(D, 1),
                       w3.reshape(D, 1))
    return _reduce_stage(cu_pad, logits, x)


# speedup vs baseline: 1.2949x; 1.2949x over previous
import functools

import jax
import jax.numpy as jnp
from jax import lax
from jax.experimental import pallas as pl
from jax.experimental.pallas import tpu as pltpu
from jax.experimental.pallas import tpu_sc as plsc

B = 1024
D = 64
T = 51200
V = 100000
NC = 2
NS = 16
NW = NC * NS
TPW = T // NW
NPW = B // NW
GC = 80
NCH = TPW // GC
NSLOT = 4
CT = 64
KB = 2048
TPAD = T + KB
CUP = 1048

_mesh = plsc.VectorSubcoreMesh(core_axis_name="c", subcore_axis_name="s")
_sc_params = pltpu.CompilerParams(needs_layout_passes=False,
                                  use_tc_tiling_on_sc=False)


def _iota16():
    return lax.broadcasted_iota(jnp.int32, (16,), 0)


@functools.partial(
    pl.kernel,
    out_type=jax.ShapeDtypeStruct((TPAD, 2 * D), jnp.float32),
    mesh=_mesh,
    compiler_params=_sc_params,
    scratch_types=[
        pltpu.VMEM((CUP,), jnp.int32),
        pltpu.VMEM((B,), jnp.int32),
        pltpu.VMEM((TPW,), jnp.int32),
        pltpu.VMEM((TPW,), jnp.int32),
        [pltpu.VMEM((GC, D), jnp.float32)] * NSLOT,
        [pltpu.VMEM((GC, D), jnp.float32)] * NSLOT,
        [pltpu.SemaphoreType.DMA] * NSLOT,
        [pltpu.SemaphoreType.DMA] * NSLOT,
        [pltpu.SemaphoreType.DMA] * NSLOT,
        [pltpu.SemaphoreType.DMA] * NSLOT,
    ],
)
def _gather_stage(cu_hbm, nodes_hbm, fn_hbm, table_hbm, out_hbm,
                  cu_v, nodes_v, fn_v, idx2_v, rows_n, rows_u,
                  gsem_n, gsem_u, wsem_n, wsem_u):
    wid = lax.axis_index("s") * NC + lax.axis_index("c")
    base = wid * TPW
    pltpu.sync_copy(cu_hbm, cu_v)
    pltpu.sync_copy(nodes_hbm, nodes_v)
    pltpu.sync_copy(fn_hbm.at[pl.ds(base, TPW)], fn_v)

    def bisect16(t, lo, hi):
        def cond(lh):
            return jnp.max(lh[1] - lh[0]) > 0

        def step(lh):
            lo_, hi_ = lh
            mid = lax.shift_right_logical(lo_ + hi_, 1)
            a = plsc.load_gather(cu_v, [mid + 1])
            p = a <= t
            return jnp.where(p, mid + 1, lo_), jnp.where(p, hi_, mid)

        lo, hi = lax.while_loop(cond, step, (lo, hi))
        return lo

    tlast = jnp.full((16,), base + TPW - 1, jnp.int32)
    hi0v = bisect16(tlast, jnp.zeros((16,), jnp.int32),
                    jnp.full((16,), B - 1, jnp.int32))
    hi0 = hi0v[0]

    def start_n(k):
        off = k * GC
        pltpu.async_copy(table_hbm.at[fn_v.at[pl.ds(off, GC)]],
                         rows_n[k % NSLOT], gsem_n[k % NSLOT])

    def start_u(k):
        off = k * GC
        pltpu.async_copy(table_hbm.at[idx2_v.at[pl.ds(off, GC)]],
                         rows_u[k % NSLOT], gsem_u[k % NSLOT])

    def wait_writes(k):
        s = k % NSLOT
        off = k * GC
        pltpu.make_async_copy(rows_n[s],
                              out_hbm.at[pl.ds(base + off, GC), pl.ds(0, D)],
                              wsem_n[s]).wait()
        pltpu.make_async_copy(rows_u[s],
                              out_hbm.at[pl.ds(base + off, GC), pl.ds(D, D)],
                              wsem_u[s]).wait()

    def finish(k):
        s = k % NSLOT
        off = k * GC
        pltpu.make_async_copy(table_hbm.at[fn_v.at[pl.ds(off, GC)]],
                              rows_n[s], gsem_n[s]).wait()
        pltpu.make_async_copy(table_hbm.at[idx2_v.at[pl.ds(off, GC)]],
                              rows_u[s], gsem_u[s]).wait()
        pltpu.async_copy(rows_n[s],
                         out_hbm.at[pl.ds(base + off, GC), pl.ds(0, D)],
                         wsem_n[s])
        pltpu.async_copy(rows_u[s],
                         out_hbm.at[pl.ds(base + off, GC), pl.ds(D, D)],
                         wsem_u[s])

    cur = jnp.int32(0)
    for k in range(NCH):
        if k >= NSLOT:
            wait_writes(k - NSLOT)
        start_n(k)
        for gg in range(GC // 16):
            t = base + (k * (GC // 16) + gg) * 16 + _iota16()
            lo = bisect16(t, jnp.full((16,), cur, jnp.int32),
                          jnp.full((16,), hi0, jnp.int32))
            idx2_v[pl.ds(k * GC + gg * 16, 16)] = (
                plsc.load_gather(nodes_v, [lo]))
            cur = lo[15]
        start_u(k)
        if k >= 1:
            finish(k - 1)
    finish(NCH - 1)
    for k in range(max(NCH - NSLOT, 0), NCH):
        wait_writes(k)


_DNT = (((0,), (1,)), ((), ()))
_DN0 = (((0,), (0,)), ((), ()))


def _mlp_body(x_ref, w1_ref, b1_ref, w2_ref, b2_ref, w3_ref, out_ref):
    bf = jnp.bfloat16
    h = lax.dot_general(w1_ref[...].astype(bf), x_ref[...].astype(bf),
                        _DNT, preferred_element_type=jnp.float32)
    h = jnp.maximum(h + b1_ref[...], 0.0)
    h = lax.dot_general(w2_ref[...].astype(bf), h.astype(bf),
                        _DN0, preferred_element_type=jnp.float32)
    h = jnp.maximum(h + b2_ref[...], 0.0)
    lg = lax.dot_general(w3_ref[...].astype(bf), h.astype(bf),
                         _DN0, preferred_element_type=jnp.float32)
    out_ref[...] = lg[0]


_mlp_call = pl.pallas_call(
    _mlp_body,
    grid=(TPAD // KB,),
    in_specs=[
        pl.BlockSpec((KB, 2 * D), lambda i: (i, 0)),
        pl.BlockSpec((2 * D, D), lambda i: (0, 0)),
        pl.BlockSpec((D, 1), lambda i: (0, 0)),
        pl.BlockSpec((D, D), lambda i: (0, 0)),
        pl.BlockSpec((D, 1), lambda i: (0, 0)),
        pl.BlockSpec((D, 1), lambda i: (0, 0)),
    ],
    out_specs=pl.BlockSpec((KB,), lambda i: (i,)),
    out_shape=jax.ShapeDtypeStruct((TPAD,), jnp.float32),
)


@functools.partial(
    pl.kernel,
    out_type=jax.ShapeDtypeStruct((B, D), jnp.float32),
    mesh=_mesh,
    compiler_params=_sc_params,
    scratch_types=[
        pltpu.VMEM((48,), jnp.int32),
        [pltpu.VMEM((CT,), jnp.float32)] * 2,
        pltpu.VMEM((CT,), jnp.float32),
        [pltpu.VMEM((CT, D), jnp.float32)] * 2,
        pltpu.VMEM((NPW, D), jnp.float32),
        pltpu.SMEM((4,), jnp.int32),
        pltpu.SMEM((4,), jnp.float32),
        pltpu.VMEM((5, 16), jnp.float32),
        [pltpu.SemaphoreType.DMA] * 2,
        [pltpu.SemaphoreType.DMA] * 2,
    ],
)
def _reduce_stage(cu_hbm, lg_hbm, en_hbm, out_hbm,
                  cu_v, lg_v, w_v, rows_v, out_v, si, sf, vs, sem_l, sem_r):
    wid = lax.axis_index("s") * NC + lax.axis_index("c")
    nbase = wid * NPW
    pltpu.sync_copy(cu_hbm.at[pl.ds(nbase, 48)], cu_v)
    neg = jnp.float32(-jnp.inf)
    zero = jnp.zeros((16,), jnp.float32)

    head = cu_v[pl.ds(0, 16)]
    tail = cu_v[pl.ds(NPW, 16)]
    s0 = head[0]
    big_e = tail[0]
    s8 = pl.multiple_of(lax.shift_left(lax.shift_right_logical(s0, 3), 3), 8)
    n_ch = lax.shift_right_logical(big_e - s8 + (CT - 1), 6)

    si[0] = 0
    si[1] = s0
    si[2] = head[1]
    sf[0] = neg
    for i in range(5):
        vs[i, pl.ds(0, 16)] = zero

    def start_dma(k, slot):
        g = pl.multiple_of(s8 + k * CT, 8)
        pltpu.async_copy(lg_hbm.at[pl.ds(g, CT)], lg_v[slot], sem_l[slot])
        pltpu.async_copy(en_hbm.at[pl.ds(g, CT), pl.ds(0, D)],
                         rows_v[slot], sem_r[slot])

    def wait_dma(k, slot):
        g = pl.multiple_of(s8 + k * CT, 8)
        pltpu.make_async_copy(lg_hbm.at[pl.ds(g, CT)], lg_v[slot],
                              sem_l[slot]).wait()
        pltpu.make_async_copy(en_hbm.at[pl.ds(g, CT), pl.ds(0, D)],
                              rows_v[slot], sem_r[slot]).wait()

    @pl.when(n_ch > 0)
    def _():
        start_dma(0, 0)

    def process(k, slot):
        g = pl.multiple_of(s8 + k * CT, 8)
        gend = g + CT

        def cond(c):
            return c[9] != 0

        def body(c):
            b, s, e, m, ssumv, a0, a1, a2, a3, _ = c
            msub = jnp.full((16,), neg, jnp.float32)
            lvs = []
            msks = []
            for q in range(CT // 16):
                gidx = g + q * 16 + _iota16()
                msk = (gidx >= s) & (gidx < e)
                lv = lg_v[slot][pl.ds(q * 16, 16)]
                lvs.append(lv)
                msks.append(msk)
                msub = jnp.maximum(msub, jnp.where(msk, lv, neg))
            mnew = jnp.maximum(m, jnp.max(msub))
            scale = jnp.exp(jnp.full((16,), m - mnew, jnp.float32))
            ssumv = ssumv * scale
            for q in range(CT // 16):
                wv = jnp.where(msks[q], jnp.exp(lvs[q] - mnew), 0.0)
                w_v[pl.ds(q * 16, 16)] = wv
                ssumv = ssumv + wv
            a0 = a0 * scale
            a1 = a1 * scale
            a2 = a2 * scale
            a3 = a3 * scale

            def tok(j, acc):
                t0, t1, t2, t3 = acc
                wj = plsc.load_gather(w_v, [jnp.full((16,), j, jnp.int32)])
                t0 = t0 + wj * rows_v[slot][j, pl.ds(0, 16)]
                t1 = t1 + wj * rows_v[slot][j, pl.ds(16, 16)]
                t2 = t2 + wj * rows_v[slot][j, pl.ds(32, 16)]
                t3 = t3 + wj * rows_v[slot][j, pl.ds(48, 16)]
                return t0, t1, t2, t3

            jlo = jnp.maximum(s - g, 0)
            jhi = jnp.minimum(e - g, CT)
            a0, a1, a2, a3 = lax.fori_loop(jlo, jhi, tok, (a0, a1, a2, a3))

            fin = e <= gend

            @pl.when(fin)
            def _():
                total = jnp.sum(ssumv)
                ok = total > 0.0
                out_v[b, pl.ds(0, 16)] = jnp.where(ok, a0 / total, 0.0)
                out_v[b, pl.ds(16, 16)] = jnp.where(ok, a1 / total, 0.0)
                out_v[b, pl.ds(32, 16)] = jnp.where(ok, a2 / total, 0.0)
                out_v[b, pl.ds(48, 16)] = jnp.where(ok, a3 / total, 0.0)

            b2 = jnp.where(fin, b + 1, b)
            pair = cu_v[pl.ds(b2, 16)]
            s2 = jnp.where(fin, pair[0], s)
            e2 = jnp.where(fin, pair[1], e)
            m2 = jnp.where(fin, neg, mnew)
            ssumv2 = jnp.where(fin, zero, ssumv)
            a02 = jnp.where(fin, zero, a0)
            a12 = jnp.where(fin, zero, a1)
            a22 = jnp.where(fin, zero, a2)
            a32 = jnp.where(fin, zero, a3)
            cont = jnp.where(fin & (b2 < NPW) & (s2 < gend),
                             jnp.int32(1), jnp.int32(0))
            return b2, s2, e2, m2, ssumv2, a02, a12, a22, a32, cont

        state = (si[0], si[1], si[2], sf[0],
                 vs[0, pl.ds(0, 16)], vs[1, pl.ds(0, 16)],
                 vs[2, pl.ds(0, 16)], vs[3, pl.ds(0, 16)],
                 vs[4, pl.ds(0, 16)], jnp.int32(1))
        b, s, e, m, ssumv, a0, a1, a2, a3, _ = lax.while_loop(
            cond, body, state)
        si[0] = b
        si[1] = s
        si[2] = e
        sf[0] = m
        vs[0, pl.ds(0, 16)] = ssumv
        vs[1, pl.ds(0, 16)] = a0
        vs[2, pl.ds(0, 16)] = a1
        vs[3, pl.ds(0, 16)] = a2
        vs[4, pl.ds(0, 16)] = a3

    def pair_body(i, _):
        k0 = i * 2
        for off in range(2):
            k = k0 + off
            slot = off

            @pl.when(k < n_ch)
            def _():
                wait_dma(k, slot)

                @pl.when(k + 1 < n_ch)
                def _():
                    start_dma(k + 1, 1 - slot)

                process(k, slot)
        return 0

    lax.fori_loop(0, lax.shift_right_logical(n_ch + 1, 1), pair_body, 0)

    def drain(b, _):
        out_v[b, pl.ds(0, 16)] = zero
        out_v[b, pl.ds(16, 16)] = zero
        out_v[b, pl.ds(32, 16)] = zero
        out_v[b, pl.ds(48, 16)] = zero
        return 0

    lax.fori_loop(si[0], NPW, drain, 0)
    pltpu.sync_copy(out_v, out_hbm.at[pl.ds(nbase, NPW)])


def kernel(nodes, flat_neighs, cu_seqlens, table, w1, b1, w2, b2, w3, b3):
    del b3
    cu_pad = jnp.full((CUP,), T, jnp.int32)
    cu_pad = lax.dynamic_update_slice(cu_pad, cu_seqlens.astype(jnp.int32),
                                      (0,))
    x = _gather_stage(cu_pad, nodes.astype(jnp.int32),
                      flat_neighs.astype(jnp.int32), table)
    logits = _mlp_call(x, w1, b1.reshape(D, 1), w2, b2.reshape(D, 1),
                       w3.reshape(D, 1))
    return _reduce_stage(cu_pad, logits, x)


# speedup vs baseline: 1.3657x; 1.0547x over previous
import functools

import jax
import jax.numpy as jnp
from jax import lax
from jax.experimental import pallas as pl
from jax.experimental.pallas import tpu as pltpu
from jax.experimental.pallas import tpu_sc as plsc

B = 1024
D = 64
T = 51200
V = 100000
NC = 2
NS = 16
NW = NC * NS
TPW = T // NW
NPW = B // NW
GC = 80
NCH = TPW // GC
NSLOT = 4
CT = 128
KB = 2048
TPAD = T + KB
CUP = 1048

_mesh = plsc.VectorSubcoreMesh(core_axis_name="c", subcore_axis_name="s")
_sc_params = pltpu.CompilerParams(needs_layout_passes=False,
                                  use_tc_tiling_on_sc=False)


def _iota16():
    return lax.broadcasted_iota(jnp.int32, (16,), 0)


@functools.partial(
    pl.kernel,
    out_type=jax.ShapeDtypeStruct((TPAD, 2 * D), jnp.float32),
    mesh=_mesh,
    compiler_params=_sc_params,
    scratch_types=[
        pltpu.VMEM((CUP,), jnp.int32),
        pltpu.VMEM((B,), jnp.int32),
        pltpu.VMEM((TPW,), jnp.int32),
        pltpu.VMEM((TPW,), jnp.int32),
        [pltpu.VMEM((GC, D), jnp.float32)] * NSLOT,
        [pltpu.VMEM((GC, D), jnp.float32)] * NSLOT,
        [pltpu.SemaphoreType.DMA] * NSLOT,
        [pltpu.SemaphoreType.DMA] * NSLOT,
        [pltpu.SemaphoreType.DMA] * NSLOT,
        [pltpu.SemaphoreType.DMA] * NSLOT,
    ],
)
def _gather_stage(cu_hbm, nodes_hbm, fn_hbm, table_hbm, out_hbm,
                  cu_v, nodes_v, fn_v, idx2_v, rows_n, rows_u,
                  gsem_n, gsem_u, wsem_n, wsem_u):
    wid = lax.axis_index("s") * NC + lax.axis_index("c")
    base = wid * TPW
    pltpu.sync_copy(cu_hbm, cu_v)
    pltpu.sync_copy(nodes_hbm, nodes_v)
    pltpu.sync_copy(fn_hbm.at[pl.ds(base, TPW)], fn_v)

    def bisect16(t, lo, hi):
        def cond(lh):
            return jnp.max(lh[1] - lh[0]) > 0

        def step(lh):
            lo_, hi_ = lh
            mid = lax.shift_right_logical(lo_ + hi_, 1)
            a = plsc.load_gather(cu_v, [mid + 1])
            p = a <= t
            return jnp.where(p, mid + 1, lo_), jnp.where(p, hi_, mid)

        lo, hi = lax.while_loop(cond, step, (lo, hi))
        return lo

    tlast = jnp.full((16,), base + TPW - 1, jnp.int32)
    hi0v = bisect16(tlast, jnp.zeros((16,), jnp.int32),
                    jnp.full((16,), B - 1, jnp.int32))
    hi0 = hi0v[0]

    def start_n(k):
        off = k * GC
        pltpu.async_copy(table_hbm.at[fn_v.at[pl.ds(off, GC)]],
                         rows_n[k % NSLOT], gsem_n[k % NSLOT])

    def start_u(k):
        off = k * GC
        pltpu.async_copy(table_hbm.at[idx2_v.at[pl.ds(off, GC)]],
                         rows_u[k % NSLOT], gsem_u[k % NSLOT])

    def wait_writes(k):
        s = k % NSLOT
        off = k * GC
        pltpu.make_async_copy(rows_n[s],
                              out_hbm.at[pl.ds(base + off, GC), pl.ds(0, D)],
                              wsem_n[s]).wait()
        pltpu.make_async_copy(rows_u[s],
                              out_hbm.at[pl.ds(base + off, GC), pl.ds(D, D)],
                              wsem_u[s]).wait()

    def finish(k):
        s = k % NSLOT
        off = k * GC
        pltpu.make_async_copy(table_hbm.at[fn_v.at[pl.ds(off, GC)]],
                              rows_n[s], gsem_n[s]).wait()
        pltpu.make_async_copy(table_hbm.at[idx2_v.at[pl.ds(off, GC)]],
                              rows_u[s], gsem_u[s]).wait()
        pltpu.async_copy(rows_n[s],
                         out_hbm.at[pl.ds(base + off, GC), pl.ds(0, D)],
                         wsem_n[s])
        pltpu.async_copy(rows_u[s],
                         out_hbm.at[pl.ds(base + off, GC), pl.ds(D, D)],
                         wsem_u[s])

    cur = jnp.int32(0)
    for k in range(NCH):
        if k >= NSLOT:
            wait_writes(k - NSLOT)
        start_n(k)
        for gg in range(GC // 16):
            t = base + (k * (GC // 16) + gg) * 16 + _iota16()
            lo = bisect16(t, jnp.full((16,), cur, jnp.int32),
                          jnp.full((16,), hi0, jnp.int32))
            idx2_v[pl.ds(k * GC + gg * 16, 16)] = (
                plsc.load_gather(nodes_v, [lo]))
            cur = lo[15]
        start_u(k)
        if k >= 1:
            finish(k - 1)
    finish(NCH - 1)
    for k in range(max(NCH - NSLOT, 0), NCH):
        wait_writes(k)


_DNT = (((0,), (1,)), ((), ()))
_DN0 = (((0,), (0,)), ((), ()))


def _mlp_body(x_ref, w1_ref, b1_ref, w2_ref, b2_ref, w3_ref, out_ref):
    bf = jnp.bfloat16
    h = lax.dot_general(w1_ref[...].astype(bf), x_ref[...].astype(bf),
                        _DNT, preferred_element_type=jnp.float32)
    h = jnp.maximum(h + b1_ref[...], 0.0)
    h = lax.dot_general(w2_ref[...].astype(bf), h.astype(bf),
                        _DN0, preferred_element_type=jnp.float32)
    h = jnp.maximum(h + b2_ref[...], 0.0)
    lg = lax.dot_general(w3_ref[...].astype(bf), h.astype(bf),
                         _DN0, preferred_element_type=jnp.float32)
    out_ref[...] = lg[0]


_mlp_call = pl.pallas_call(
    _mlp_body,
    grid=(TPAD // KB,),
    in_specs=[
        pl.BlockSpec((KB, 2 * D), lambda i: (i, 0)),
        pl.BlockSpec((2 * D, D), lambda i: (0, 0)),
        pl.BlockSpec((D, 1), lambda i: (0, 0)),
        pl.BlockSpec((D, D), lambda i: (0, 0)),
        pl.BlockSpec((D, 1), lambda i: (0, 0)),
        pl.BlockSpec((D, 1), lambda i: (0, 0)),
    ],
    out_specs=pl.BlockSpec((KB,), lambda i: (i,)),
    out_shape=jax.ShapeDtypeStruct((TPAD,), jnp.float32),
)


@functools.partial(
    pl.kernel,
    out_type=jax.ShapeDtypeStruct((B, D), jnp.float32),
    mesh=_mesh,
    compiler_params=_sc_params,
    scratch_types=[
        pltpu.VMEM((48,), jnp.int32),
        [pltpu.VMEM((CT,), jnp.float32)] * 2,
        pltpu.VMEM((CT,), jnp.float32),
        [pltpu.VMEM((CT, D), jnp.float32)] * 2,
        pltpu.VMEM((NPW, D), jnp.float32),
        pltpu.SMEM((4,), jnp.int32),
        pltpu.SMEM((4,), jnp.float32),
        pltpu.VMEM((5, 16), jnp.float32),
        [pltpu.SemaphoreType.DMA] * 2,
        [pltpu.SemaphoreType.DMA] * 2,
    ],
)
def _reduce_stage(cu_hbm, lg_hbm, en_hbm, out_hbm,
                  cu_v, lg_v, w_v, rows_v, out_v, si, sf, vs, sem_l, sem_r):
    wid = lax.axis_index("s") * NC + lax.axis_index("c")
    nbase = wid * NPW
    pltpu.sync_copy(cu_hbm.at[pl.ds(nbase, 48)], cu_v)
    neg = jnp.float32(-jnp.inf)
    zero = jnp.zeros((16,), jnp.float32)

    head = cu_v[pl.ds(0, 16)]
    tail = cu_v[pl.ds(NPW, 16)]
    s0 = head[0]
    big_e = tail[0]
    s8 = pl.multiple_of(lax.shift_left(lax.shift_right_logical(s0, 3), 3), 8)
    n_ch = lax.shift_right_logical(big_e - s8 + (CT - 1), 7)

    si[0] = 0
    si[1] = s0
    si[2] = head[1]
    sf[0] = neg
    for i in range(5):
        vs[i, pl.ds(0, 16)] = zero

    def start_dma(k, slot):
        g = pl.multiple_of(s8 + k * CT, 8)
        pltpu.async_copy(lg_hbm.at[pl.ds(g, CT)], lg_v[slot], sem_l[slot])
        pltpu.async_copy(en_hbm.at[pl.ds(g, CT), pl.ds(0, D)],
                         rows_v[slot], sem_r[slot])

    def wait_dma(k, slot):
        g = pl.multiple_of(s8 + k * CT, 8)
        pltpu.make_async_copy(lg_hbm.at[pl.ds(g, CT)], lg_v[slot],
                              sem_l[slot]).wait()
        pltpu.make_async_copy(en_hbm.at[pl.ds(g, CT), pl.ds(0, D)],
                              rows_v[slot], sem_r[slot]).wait()

    @pl.when(n_ch > 0)
    def _():
        start_dma(0, 0)

    def process(k, slot):
        g = pl.multiple_of(s8 + k * CT, 8)
        gend = g + CT

        def cond(c):
            return c[9] != 0

        def body(c):
            b, s, e, m, ssumv, a0, a1, a2, a3, _ = c
            msub = jnp.full((16,), neg, jnp.float32)
            lvs = []
            msks = []
            for q in range(CT // 16):
                gidx = g + q * 16 + _iota16()
                msk = (gidx >= s) & (gidx < e)
                lv = lg_v[slot][pl.ds(q * 16, 16)]
                lvs.append(lv)
                msks.append(msk)
                msub = jnp.maximum(msub, jnp.where(msk, lv, neg))
            mnew = jnp.maximum(m, jnp.max(msub))
            scale = jnp.exp(jnp.full((16,), m - mnew, jnp.float32))
            ssumv = ssumv * scale
            for q in range(CT // 16):
                wv = jnp.where(msks[q], jnp.exp(lvs[q] - mnew), 0.0)
                w_v[pl.ds(q * 16, 16)] = wv
                ssumv = ssumv + wv
            a0 = a0 * scale
            a1 = a1 * scale
            a2 = a2 * scale
            a3 = a3 * scale

            def tok(j, acc):
                t0, t1, t2, t3 = acc
                wj = plsc.load_gather(w_v, [jnp.full((16,), j, jnp.int32)])
                t0 = t0 + wj * rows_v[slot][j, pl.ds(0, 16)]
                t1 = t1 + wj * rows_v[slot][j, pl.ds(16, 16)]
                t2 = t2 + wj * rows_v[slot][j, pl.ds(32, 16)]
                t3 = t3 + wj * rows_v[slot][j, pl.ds(48, 16)]
                return t0, t1, t2, t3

            jlo = jnp.maximum(s - g, 0)
            jhi = jnp.minimum(e - g, CT)
            a0, a1, a2, a3 = lax.fori_loop(jlo, jhi, tok, (a0, a1, a2, a3))

            fin = e <= gend

            @pl.when(fin)
            def _():
                total = jnp.sum(ssumv)
                ok = total > 0.0
                out_v[b, pl.ds(0, 16)] = jnp.where(ok, a0 / total, 0.0)
                out_v[b, pl.ds(16, 16)] = jnp.where(ok, a1 / total, 0.0)
                out_v[b, pl.ds(32, 16)] = jnp.where(ok, a2 / total, 0.0)
                out_v[b, pl.ds(48, 16)] = jnp.where(ok, a3 / total, 0.0)

            b2 = jnp.where(fin, b + 1, b)
            pair = cu_v[pl.ds(b2, 16)]
            s2 = jnp.where(fin, pair[0], s)
            e2 = jnp.where(fin, pair[1], e)
            m2 = jnp.where(fin, neg, mnew)
            ssumv2 = jnp.where(fin, zero, ssumv)
            a02 = jnp.where(fin, zero, a0)
            a12 = jnp.where(fin, zero, a1)
            a22 = jnp.where(fin, zero, a2)
            a32 = jnp.where(fin, zero, a3)
            cont = jnp.where(fin & (b2 < NPW) & (s2 < gend),
                             jnp.int32(1), jnp.int32(0))
            return b2, s2, e2, m2, ssumv2, a02, a12, a22, a32, cont

        state = (si[0], si[1], si[2], sf[0],
                 vs[0, pl.ds(0, 16)], vs[1, pl.ds(0, 16)],
                 vs[2, pl.ds(0, 16)], vs[3, pl.ds(0, 16)],
                 vs[4, pl.ds(0, 16)], jnp.int32(1))
        b, s, e, m, ssumv, a0, a1, a2, a3, _ = lax.while_loop(
            cond, body, state)
        si[0] = b
        si[1] = s
        si[2] = e
        sf[0] = m
        vs[0, pl.ds(0, 16)] = ssumv
        vs[1, pl.ds(0, 16)] = a0
        vs[2, pl.ds(0, 16)] = a1
        vs[3, pl.ds(0, 16)] = a2
        vs[4, pl.ds(0, 16)] = a3

    def pair_body(i, _):
        k0 = i * 2
        for off in range(2):
            k = k0 + off
            slot = off

            @pl.when(k < n_ch)
            def _():
                wait_dma(k, slot)

                @pl.when(k + 1 < n_ch)
                def _():
                    start_dma(k + 1, 1 - slot)

                process(k, slot)
        return 0

    lax.fori_loop(0, lax.shift_right_logical(n_ch + 1, 1), pair_body, 0)

    def drain(b, _):
        out_v[b, pl.ds(0, 16)] = zero
        out_v[b, pl.ds(16, 16)] = zero
        out_v[b, pl.ds(32, 16)] = zero
        out_v[b, pl.ds(48, 16)] = zero
        return 0

    lax.fori_loop(si[0], NPW, drain, 0)
    pltpu.sync_copy(out_v, out_hbm.at[pl.ds(nbase, NPW)])


def kernel(nodes, flat_neighs, cu_seqlens, table, w1, b1, w2, b2, w3, b3):
    del b3
    cu_pad = jnp.full((CUP,), T, jnp.int32)
    cu_pad = lax.dynamic_update_slice(cu_pad, cu_seqlens.astype(jnp.int32),
                                      (0,))
    x = _gather_stage(cu_pad, nodes.astype(jnp.int32),
                      flat_neighs.astype(jnp.int32), table)
    logits = _mlp_call(x, w1, b1.reshape(D, 1), w2, b2.reshape(D, 1),
                       w3.reshape(D, 1))
    return _reduce_stage(cu_pad, logits, x)


# speedup vs baseline: 1.3834x; 1.0129x over previous
import functools

import jax
import jax.numpy as jnp
from jax import lax
from jax.experimental import pallas as pl
from jax.experimental.pallas import tpu as pltpu
from jax.experimental.pallas import tpu_sc as plsc

B = 1024
D = 64
T = 51200
V = 100000
NC = 2
NS = 16
NW = NC * NS
TPW = T // NW
NPW = B // NW
GC = 80
NCH = TPW // GC
NSLOT = 4
CT = 256
KB = 2048
TPAD = T + KB
CUP = 1048

_mesh = plsc.VectorSubcoreMesh(core_axis_name="c", subcore_axis_name="s")
_sc_params = pltpu.CompilerParams(needs_layout_passes=False,
                                  use_tc_tiling_on_sc=False)


def _iota16():
    return lax.broadcasted_iota(jnp.int32, (16,), 0)


@functools.partial(
    pl.kernel,
    out_type=jax.ShapeDtypeStruct((TPAD, 2 * D), jnp.float32),
    mesh=_mesh,
    compiler_params=_sc_params,
    scratch_types=[
        pltpu.VMEM((CUP,), jnp.int32),
        pltpu.VMEM((B,), jnp.int32),
        pltpu.VMEM((TPW,), jnp.int32),
        pltpu.VMEM((TPW,), jnp.int32),
        [pltpu.VMEM((GC, D), jnp.float32)] * NSLOT,
        [pltpu.VMEM((GC, D), jnp.float32)] * NSLOT,
        [pltpu.SemaphoreType.DMA] * NSLOT,
        [pltpu.SemaphoreType.DMA] * NSLOT,
        [pltpu.SemaphoreType.DMA] * NSLOT,
        [pltpu.SemaphoreType.DMA] * NSLOT,
    ],
)
def _gather_stage(cu_hbm, nodes_hbm, fn_hbm, table_hbm, out_hbm,
                  cu_v, nodes_v, fn_v, idx2_v, rows_n, rows_u,
                  gsem_n, gsem_u, wsem_n, wsem_u):
    wid = lax.axis_index("s") * NC + lax.axis_index("c")
    base = wid * TPW
    pltpu.sync_copy(cu_hbm, cu_v)
    pltpu.sync_copy(nodes_hbm, nodes_v)
    pltpu.sync_copy(fn_hbm.at[pl.ds(base, TPW)], fn_v)

    def bisect16(t, lo, hi):
        def cond(lh):
            return jnp.max(lh[1] - lh[0]) > 0

        def step(lh):
            lo_, hi_ = lh
            mid = lax.shift_right_logical(lo_ + hi_, 1)
            a = plsc.load_gather(cu_v, [mid + 1])
            p = a <= t
            return jnp.where(p, mid + 1, lo_), jnp.where(p, hi_, mid)

        lo, hi = lax.while_loop(cond, step, (lo, hi))
        return lo

    tlast = jnp.full((16,), base + TPW - 1, jnp.int32)
    hi0v = bisect16(tlast, jnp.zeros((16,), jnp.int32),
                    jnp.full((16,), B - 1, jnp.int32))
    hi0 = hi0v[0]

    def start_n(k):
        off = k * GC
        pltpu.async_copy(table_hbm.at[fn_v.at[pl.ds(off, GC)]],
                         rows_n[k % NSLOT], gsem_n[k % NSLOT])

    def start_u(k):
        off = k * GC
        pltpu.async_copy(table_hbm.at[idx2_v.at[pl.ds(off, GC)]],
                         rows_u[k % NSLOT], gsem_u[k % NSLOT])

    def wait_writes(k):
        s = k % NSLOT
        off = k * GC
        pltpu.make_async_copy(rows_n[s],
                              out_hbm.at[pl.ds(base + off, GC), pl.ds(0, D)],
                              wsem_n[s]).wait()
        pltpu.make_async_copy(rows_u[s],
                              out_hbm.at[pl.ds(base + off, GC), pl.ds(D, D)],
                              wsem_u[s]).wait()

    def finish(k):
        s = k % NSLOT
        off = k * GC
        pltpu.make_async_copy(table_hbm.at[fn_v.at[pl.ds(off, GC)]],
                              rows_n[s], gsem_n[s]).wait()
        pltpu.make_async_copy(table_hbm.at[idx2_v.at[pl.ds(off, GC)]],
                              rows_u[s], gsem_u[s]).wait()
        pltpu.async_copy(rows_n[s],
                         out_hbm.at[pl.ds(base + off, GC), pl.ds(0, D)],
                         wsem_n[s])
        pltpu.async_copy(rows_u[s],
                         out_hbm.at[pl.ds(base + off, GC), pl.ds(D, D)],
                         wsem_u[s])

    cur = jnp.int32(0)
    for k in range(NCH):
        if k >= NSLOT:
            wait_writes(k - NSLOT)
        start_n(k)
        for gg in range(GC // 16):
            t = base + (k * (GC // 16) + gg) * 16 + _iota16()
            lo = bisect16(t, jnp.full((16,), cur, jnp.int32),
                          jnp.full((16,), hi0, jnp.int32))
            idx2_v[pl.ds(k * GC + gg * 16, 16)] = (
                plsc.load_gather(nodes_v, [lo]))
            cur = lo[15]
        start_u(k)
        if k >= 1:
            finish(k - 1)
    finish(NCH - 1)
    for k in range(max(NCH - NSLOT, 0), NCH):
        wait_writes(k)


_DNT = (((0,), (1,)), ((), ()))
_DN0 = (((0,), (0,)), ((), ()))


def _mlp_body(x_ref, w1_ref, b1_ref, w2_ref, b2_ref, w3_ref, out_ref):
    bf = jnp.bfloat16
    h = lax.dot_general(w1_ref[...].astype(bf), x_ref[...].astype(bf),
                        _DNT, preferred_element_type=jnp.float32)
    h = jnp.maximum(h + b1_ref[...], 0.0)
    h = lax.dot_general(w2_ref[...].astype(bf), h.astype(bf),
                        _DN0, preferred_element_type=jnp.float32)
    h = jnp.maximum(h + b2_ref[...], 0.0)
    lg = lax.dot_general(w3_ref[...].astype(bf), h.astype(bf),
                         _DN0, preferred_element_type=jnp.float32)
    out_ref[...] = lg[0]


_mlp_call = pl.pallas_call(
    _mlp_body,
    grid=(TPAD // KB,),
    in_specs=[
        pl.BlockSpec((KB, 2 * D), lambda i: (i, 0)),
        pl.BlockSpec((2 * D, D), lambda i: (0, 0)),
        pl.BlockSpec((D, 1), lambda i: (0, 0)),
        pl.BlockSpec((D, D), lambda i: (0, 0)),
        pl.BlockSpec((D, 1), lambda i: (0, 0)),
        pl.BlockSpec((D, 1), lambda i: (0, 0)),
    ],
    out_specs=pl.BlockSpec((KB,), lambda i: (i,)),
    out_shape=jax.ShapeDtypeStruct((TPAD,), jnp.float32),
)


@functools.partial(
    pl.kernel,
    out_type=jax.ShapeDtypeStruct((B, D), jnp.float32),
    mesh=_mesh,
    compiler_params=_sc_params,
    scratch_types=[
        pltpu.VMEM((48,), jnp.int32),
        [pltpu.VMEM((CT,), jnp.float32)] * 2,
        pltpu.VMEM((CT,), jnp.float32),
        [pltpu.VMEM((CT, D), jnp.float32)] * 2,
        pltpu.VMEM((NPW, D), jnp.float32),
        pltpu.SMEM((4,), jnp.int32),
        pltpu.SMEM((4,), jnp.float32),
        pltpu.VMEM((5, 16), jnp.float32),
        [pltpu.SemaphoreType.DMA] * 2,
        [pltpu.SemaphoreType.DMA] * 2,
    ],
)
def _reduce_stage(cu_hbm, lg_hbm, en_hbm, out_hbm,
                  cu_v, lg_v, w_v, rows_v, out_v, si, sf, vs, sem_l, sem_r):
    wid = lax.axis_index("s") * NC + lax.axis_index("c")
    nbase = wid * NPW
    pltpu.sync_copy(cu_hbm.at[pl.ds(nbase, 48)], cu_v)
    neg = jnp.float32(-jnp.inf)
    zero = jnp.zeros((16,), jnp.float32)

    head = cu_v[pl.ds(0, 16)]
    tail = cu_v[pl.ds(NPW, 16)]
    s0 = head[0]
    big_e = tail[0]
    s8 = pl.multiple_of(lax.shift_left(lax.shift_right_logical(s0, 3), 3), 8)
    n_ch = lax.shift_right_logical(big_e - s8 + (CT - 1), 8)

    si[0] = 0
    si[1] = s0
    si[2] = head[1]
    sf[0] = neg
    for i in range(5):
        vs[i, pl.ds(0, 16)] = zero

    def start_dma(k, slot):
        g = pl.multiple_of(s8 + k * CT, 8)
        pltpu.async_copy(lg_hbm.at[pl.ds(g, CT)], lg_v[slot], sem_l[slot])
        pltpu.async_copy(en_hbm.at[pl.ds(g, CT), pl.ds(0, D)],
                         rows_v[slot], sem_r[slot])

    def wait_dma(k, slot):
        g = pl.multiple_of(s8 + k * CT, 8)
        pltpu.make_async_copy(lg_hbm.at[pl.ds(g, CT)], lg_v[slot],
                              sem_l[slot]).wait()
        pltpu.make_async_copy(en_hbm.at[pl.ds(g, CT), pl.ds(0, D)],
                              rows_v[slot], sem_r[slot]).wait()

    @pl.when(n_ch > 0)
    def _():
        start_dma(0, 0)

    def process(k, slot):
        g = pl.multiple_of(s8 + k * CT, 8)
        gend = g + CT

        def cond(c):
            return c[9] != 0

        def body(c):
            b, s, e, m, ssumv, a0, a1, a2, a3, _ = c
            msub = jnp.full((16,), neg, jnp.float32)
            lvs = []
            msks = []
            for q in range(CT // 16):
                gidx = g + q * 16 + _iota16()
                msk = (gidx >= s) & (gidx < e)
                lv = lg_v[slot][pl.ds(q * 16, 16)]
                lvs.append(lv)
                msks.append(msk)
                msub = jnp.maximum(msub, jnp.where(msk, lv, neg))
            mnew = jnp.maximum(m, jnp.max(msub))
            scale = jnp.exp(jnp.full((16,), m - mnew, jnp.float32))
            ssumv = ssumv * scale
            for q in range(CT // 16):
                wv = jnp.where(msks[q], jnp.exp(lvs[q] - mnew), 0.0)
                w_v[pl.ds(q * 16, 16)] = wv
                ssumv = ssumv + wv
            a0 = a0 * scale
            a1 = a1 * scale
            a2 = a2 * scale
            a3 = a3 * scale

            def tok(j, acc):
                t0, t1, t2, t3 = acc
                wj = plsc.load_gather(w_v, [jnp.full((16,), j, jnp.int32)])
                t0 = t0 + wj * rows_v[slot][j, pl.ds(0, 16)]
                t1 = t1 + wj * rows_v[slot][j, pl.ds(16, 16)]
                t2 = t2 + wj * rows_v[slot][j, pl.ds(32, 16)]
                t3 = t3 + wj * rows_v[slot][j, pl.ds(48, 16)]
                return t0, t1, t2, t3

            jlo = jnp.maximum(s - g, 0)
            jhi = jnp.minimum(e - g, CT)
            a0, a1, a2, a3 = lax.fori_loop(jlo, jhi, tok, (a0, a1, a2, a3))

            fin = e <= gend

            @pl.when(fin)
            def _():
                total = jnp.sum(ssumv)
                ok = total > 0.0
                out_v[b, pl.ds(0, 16)] = jnp.where(ok, a0 / total, 0.0)
                out_v[b, pl.ds(16, 16)] = jnp.where(ok, a1 / total, 0.0)
                out_v[b, pl.ds(32, 16)] = jnp.where(ok, a2 / total, 0.0)
                out_v[b, pl.ds(48, 16)] = jnp.where(ok, a3 / total, 0.0)

            b2 = jnp.where(fin, b + 1, b)
            pair = cu_v[pl.ds(b2, 16)]
            s2 = jnp.where(fin, pair[0], s)
            e2 = jnp.where(fin, pair[1], e)
            m2 = jnp.where(fin, neg, mnew)
            ssumv2 = jnp.where(fin, zero, ssumv)
            a02 = jnp.where(fin, zero, a0)
            a12 = jnp.where(fin, zero, a1)
            a22 = jnp.where(fin, zero, a2)
            a32 = jnp.where(fin, zero, a3)
            cont = jnp.where(fin & (b2 < NPW) & (s2 < gend),
                             jnp.int32(1), jnp.int32(0))
            return b2, s2, e2, m2, ssumv2, a02, a12, a22, a32, cont

        state = (si[0], si[1], si[2], sf[0],
                 vs[0, pl.ds(0, 16)], vs[1, pl.ds(0, 16)],
                 vs[2, pl.ds(0, 16)], vs[3, pl.ds(0, 16)],
                 vs[4, pl.ds(0, 16)], jnp.int32(1))
        b, s, e, m, ssumv, a0, a1, a2, a3, _ = lax.while_loop(
            cond, body, state)
        si[0] = b
        si[1] = s
        si[2] = e
        sf[0] = m
        vs[0, pl.ds(0, 16)] = ssumv
        vs[1, pl.ds(0, 16)] = a0
        vs[2, pl.ds(0, 16)] = a1
        vs[3, pl.ds(0, 16)] = a2
        vs[4, pl.ds(0, 16)] = a3

    def pair_body(i, _):
        k0 = i * 2
        for off in range(2):
            k = k0 + off
            slot = off

            @pl.when(k < n_ch)
            def _():
                wait_dma(k, slot)

                @pl.when(k + 1 < n_ch)
                def _():
                    start_dma(k + 1, 1 - slot)

                process(k, slot)
        return 0

    lax.fori_loop(0, lax.shift_right_logical(n_ch + 1, 1), pair_body, 0)

    def drain(b, _):
        out_v[b, pl.ds(0, 16)] = zero
        out_v[b, pl.ds(16, 16)] = zero
        out_v[b, pl.ds(32, 16)] = zero
        out_v[b, pl.ds(48, 16)] = zero
        return 0

    lax.fori_loop(si[0], NPW, drain, 0)
    pltpu.sync_copy(out_v, out_hbm.at[pl.ds(nbase, NPW)])


def kernel(nodes, flat_neighs, cu_seqlens, table, w1, b1, w2, b2, w3, b3):
    del b3
    cu_pad = jnp.full((CUP,), T, jnp.int32)
    cu_pad = lax.dynamic_update_slice(cu_pad, cu_seqlens.astype(jnp.int32),
                                      (0,))
    x = _gather_stage(cu_pad, nodes.astype(jnp.int32),
                      flat_neighs.astype(jnp.int32), table)
    logits = _mlp_call(x, w1, b1.reshape(D, 1), w2, b2.reshape(D, 1),
                       w3.reshape(D, 1))
    return _reduce_stage(cu_pad, logits, x)
